# baseline (speedup 1.0000x reference)
import functools
import math

import jax
import jax.numpy as jnp
import numpy as np
from jax.experimental import pallas as pl
from jax.experimental.pallas import tpu as pltpu

_N = 524288
_GRID_LEVELS = 16
_GRID_F = 2
_GRID_LOG2_T = 19
_GRID_BASE = 16
_MAX_RES = 1024
_GRID_PLS = float(np.exp((np.log(_MAX_RES) - np.log(_GRID_BASE)) / (_GRID_LEVELS - 1)))
_PLANE_LEVELS = 4
_PLANE_LOG2_T = 17
_PLANE_BASE = _MAX_RES // 4
_POS_DEG = 4
_PRIMES = (1, 2654435761, 805459861)


def _enc_hashgrid(x, table, n_levels, base_res, scale, log2_T):
    T = 2 ** log2_T
    d = x.shape[-1]
    outs = []
    for l in range(n_levels):
        res = int(np.floor(base_res * (scale ** l)))
        pos = x * res
        pos_floor = jnp.floor(pos)
        frac = pos - pos_floor
        c0 = pos_floor.astype(jnp.uint32)
        acc = jnp.zeros((x.shape[0], table.shape[-1]), dtype=x.dtype)
        dense = (res + 1) ** d <= T
        for corner in range(2 ** d):
            bits = [(corner >> i) & 1 for i in range(d)]
            c = c0 + jnp.asarray(np.array(bits, dtype=np.uint32))
            if dense:
                idx = c[:, 0]
                stride = res + 1
                for i in range(1, d):
                    idx = idx + c[:, i] * np.uint32(stride)
                    stride = stride * (res + 1)
            else:
                idx = c[:, 0] * np.uint32(_PRIMES[0])
                for i in range(1, d):
                    idx = jnp.bitwise_xor(idx, c[:, i] * np.uint32(_PRIMES[i]))
                idx = jnp.bitwise_and(idx, np.uint32(T - 1))
            w = jnp.ones((x.shape[0],), dtype=x.dtype)
            for i in range(d):
                w = w * (frac[:, i] if bits[i] else (1.0 - frac[:, i]))
            acc = acc + w[:, None] * jnp.take(table[l], idx.astype(jnp.int32), axis=0)
        outs.append(acc)
    return jnp.concatenate(outs, axis=-1)


def _mlp_body(enc_ref, w0_ref, w1_ref, w2_ref, out_ref):
    h = jnp.dot(enc_ref[...], w0_ref[...], preferred_element_type=jnp.float32)
    h = jnp.maximum(h, 0.0)
    h = jnp.dot(h, w1_ref[...], preferred_element_type=jnp.float32)
    h = jnp.maximum(h, 0.0)
    out_ref[...] = jnp.dot(h, w2_ref[...], preferred_element_type=jnp.float32)


@jax.jit
def _mlp(enc, W0, W1, W2):
    n, e = enc.shape
    B = 4096
    return pl.pallas_call(
        _mlp_body,
        grid=(n // B,),
        in_specs=[
            pl.BlockSpec((B, e), lambda i: (i, 0)),
            pl.BlockSpec(W0.shape, lambda i: (0, 0)),
            pl.BlockSpec(W1.shape, lambda i: (0, 0)),
            pl.BlockSpec(W2.shape, lambda i: (0, 0)),
        ],
        out_specs=pl.BlockSpec((B, W2.shape[1]), lambda i: (i, 0)),
        out_shape=jax.ShapeDtypeStruct((n, W2.shape[1]), jnp.float32),
    )(enc, W0, W1, W2)


def kernel(in_tensor, grid_table, plane0, plane1, plane2, W0, W1, W2):
    g = _enc_hashgrid(in_tensor, grid_table, _GRID_LEVELS, _GRID_BASE, _GRID_PLS, _GRID_LOG2_T)
    p0 = _enc_hashgrid(in_tensor[:, np.array([0, 1])], plane0, _PLANE_LEVELS, _PLANE_BASE, 2.0, _PLANE_LOG2_T)
    p1 = _enc_hashgrid(in_tensor[:, np.array([1, 2])], plane1, _PLANE_LEVELS, _PLANE_BASE, 2.0, _PLANE_LOG2_T)
    p2 = _enc_hashgrid(in_tensor[:, np.array([2, 0])], plane2, _PLANE_LEVELS, _PLANE_BASE, 2.0, _PLANE_LOG2_T)
    scales = jnp.asarray([2.0 ** i for i in range(_POS_DEG)], dtype=jnp.float32)
    xb = (in_tensor[..., None, :] * scales[:, None]).reshape(in_tensor.shape[0], _POS_DEG * 3)
    pe = jnp.sin(jnp.concatenate([xb, xb + 0.5 * math.pi], axis=-1))
    enc = jnp.concatenate([g, p0, p1, p2, pe], axis=-1)
    return _mlp(enc, W0, W1, W2)


# speedup vs baseline: 20.9511x; 20.9511x over previous
import functools
import math

import jax
import jax.numpy as jnp
import numpy as np
from jax import lax
from jax.experimental import pallas as pl
from jax.experimental.pallas import tpu as pltpu
import jax.experimental.pallas.tpu_sc as plsc

_N = 524288
_GRID_LEVELS = 16
_GRID_LOG2_T = 19
_GRID_BASE = 16
_MAX_RES = 1024
_GRID_PLS = float(np.exp((np.log(_MAX_RES) - np.log(_GRID_BASE)) / (_GRID_LEVELS - 1)))
_PLANE_LOG2_T = 17
_PLANE_BASE = _MAX_RES // 4
_POS_DEG = 4
_P1 = np.int32(np.uint32(2654435761).astype(np.int32))
_P2 = np.int32(805459861)

_NW = 32
_PPW = _N // _NW
_C = 512
_STEPS = _C // 16
_NCH = _PPW // _C
_SUB = 2048

_GRID_RES = [int(np.floor(_GRID_BASE * (_GRID_PLS ** l))) for l in range(_GRID_LEVELS)]
_GRID_T = 1 << _GRID_LOG2_T
_PLANE_T = 1 << _PLANE_LOG2_T
_PLANE_RES = [_PLANE_BASE * (2 ** q) for q in range(4)]
_FLAT_GRID_OFF = 0
_FLAT_PLANE_OFF = _GRID_LEVELS * _GRID_T


def _emit_grid_level(l, xa, xb, xc, res):
    S = res + 1
    dense = S ** 3 <= _GRID_T
    row_off = l * _GRID_T

    def indices(j0, xav, xbv, xcv):
        rf = jnp.float32(res)
        fa = xav * rf
        fb = xbv * rf
        fc = xcv * rf
        ia = fa.astype(jnp.int32)
        ib = fb.astype(jnp.int32)
        ic = fc.astype(jnp.int32)
        out = []
        if dense:
            base = ia + ib * np.int32(S) + ic * np.int32(S * S)
            e = base * np.int32(2) + np.int32(2 * row_off)
            for k in range(8):
                b0, b1, b2 = k & 1, (k >> 1) & 1, (k >> 2) & 1
                off = 2 * (b0 + b1 * S + b2 * S * S)
                e0 = e + np.int32(off)
                out.append((e0, e0 + np.int32(1)))
        else:
            h1 = ib * _P1
            h1p = h1 + _P1
            h2 = ic * _P2
            h2p = h2 + _P2
            iap = ia + np.int32(1)
            msk = np.int32(_GRID_T - 1)
            for k in range(8):
                b0, b1, b2 = k & 1, (k >> 1) & 1, (k >> 2) & 1
                t = (iap if b0 else ia) ^ (h1p if b1 else h1) ^ (h2p if b2 else h2)
                m = t & msk
                e0 = m * np.int32(2) + np.int32(2 * row_off)
                out.append((e0, e0 + np.int32(1)))
        return out

    def weights(xav, xbv, xcv):
        rf = jnp.float32(res)
        fa = xav * rf
        fb = xbv * rf
        fc = xcv * rf
        fra = fa - fa.astype(jnp.int32).astype(jnp.float32)
        frb = fb - fb.astype(jnp.int32).astype(jnp.float32)
        frc = fc - fc.astype(jnp.int32).astype(jnp.float32)
        wa = (jnp.float32(1.0) - fra, fra)
        wb = (jnp.float32(1.0) - frb, frb)
        wc = (jnp.float32(1.0) - frc, frc)
        ws = []
        for k in range(8):
            b0, b1, b2 = k & 1, (k >> 1) & 1, (k >> 2) & 1
            ws.append(wa[b0] * wb[b1] * wc[b2])
        return ws

    return indices, weights


def _emit_plane_level(p, q):
    res = _PLANE_RES[q]
    S = res + 1
    dense = S * S <= _PLANE_T
    row_off = _FLAT_PLANE_OFF + (p * 4 + q) * _PLANE_T

    def indices(xav, xbv):
        rf = jnp.float32(res)
        fa = xav * rf
        fb = xbv * rf
        ia = fa.astype(jnp.int32)
        ib = fb.astype(jnp.int32)
        out = []
        if dense:
            base = ia + ib * np.int32(S)
            e = base * np.int32(2) + np.int32(2 * row_off)
            for k in range(4):
                b0, b1 = k & 1, (k >> 1) & 1
                off = 2 * (b0 + b1 * S)
                e0 = e + np.int32(off)
                out.append((e0, e0 + np.int32(1)))
        else:
            h1 = ib * _P1
            h1p = h1 + _P1
            iap = ia + np.int32(1)
            msk = np.int32(_PLANE_T - 1)
            for k in range(4):
                b0, b1 = k & 1, (k >> 1) & 1
                t = (iap if b0 else ia) ^ (h1p if b1 else h1)
                m = t & msk
                e0 = m * np.int32(2) + np.int32(2 * row_off)
                out.append((e0, e0 + np.int32(1)))
        return out

    def weights(xav, xbv):
        rf = jnp.float32(res)
        fa = xav * rf
        fb = xbv * rf
        fra = fa - fa.astype(jnp.int32).astype(jnp.float32)
        frb = fb - fb.astype(jnp.int32).astype(jnp.float32)
        wa = (jnp.float32(1.0) - fra, fra)
        wb = (jnp.float32(1.0) - frb, frb)
        return [wa[k & 1] * wb[(k >> 1) & 1] for k in range(4)]

    return indices, weights


@functools.cache
def _make_sc_encode():
    mesh = plsc.VectorSubcoreMesh(
        core_axis_name="c", subcore_axis_name="s", num_cores=2, num_subcores=16)
    return pl.kernel(
        _sc_encode_body,
        out_type=jax.ShapeDtypeStruct((56, _N), jnp.float32),
        mesh=mesh,
        scratch_types=[
            pltpu.VMEM((_C,), jnp.float32),
            pltpu.VMEM((_C,), jnp.float32),
            pltpu.VMEM((_C,), jnp.float32),
            pltpu.VMEM((16 * _C,), jnp.int32),
            pltpu.VMEM((16 * _C,), jnp.float32),
            pltpu.VMEM((56, _C), jnp.float32),
            pltpu.SemaphoreType.DMA,
        ],
    )


def _sc_encode_body(x0_hbm, x1_hbm, x2_hbm, flat_hbm, enc_hbm,
                    xa_v, xb_v, xc_v, idx_v, dst_v, acc_v, sem):
    wid = lax.axis_index("s") * 2 + lax.axis_index("c")

    def chunk(g, _):
        base = wid * _PPW + g * _C
        pltpu.sync_copy(x0_hbm.at[pl.ds(base, _C)], xa_v)
        pltpu.sync_copy(x1_hbm.at[pl.ds(base, _C)], xb_v)
        pltpu.sync_copy(x2_hbm.at[pl.ds(base, _C)], xc_v)

        def do_level(row0, xrefs, idx_fn, w_fn, ncorner):
            nidx = 2 * ncorner * _C

            def p1(step, _):
                j0 = step * 16
                xs = [r[pl.ds(j0, 16)] for r in xrefs]
                pairs = idx_fn(*xs)
                for k, (e0, e1) in enumerate(pairs):
                    idx_v[pl.ds((2 * k) * _C + j0, 16)] = e0
                    idx_v[pl.ds((2 * k + 1) * _C + j0, 16)] = e1
                return ()

            lax.fori_loop(0, _STEPS, p1, ())
            for u in range(nidx // _SUB):
                pltpu.async_copy(
                    flat_hbm.at[idx_v.at[pl.ds(u * _SUB, _SUB)]],
                    dst_v.at[pl.ds(u * _SUB, _SUB)], sem).wait()

            def p2(step, _):
                j0 = step * 16
                xs = [r[pl.ds(j0, 16)] for r in xrefs]
                ws = w_fn(*xs)
                acc0 = None
                acc1 = None
                for k in range(ncorner):
                    f0 = dst_v[pl.ds((2 * k) * _C + j0, 16)]
                    f1 = dst_v[pl.ds((2 * k + 1) * _C + j0, 16)]
                    w = ws[k]
                    if acc0 is None:
                        acc0 = w * f0
                        acc1 = w * f1
                    else:
                        acc0 = acc0 + w * f0
                        acc1 = acc1 + w * f1
                acc_v[row0, pl.ds(j0, 16)] = acc0
                acc_v[row0 + 1, pl.ds(j0, 16)] = acc1
                return ()

            lax.fori_loop(0, _STEPS, p2, ())

        for l in range(_GRID_LEVELS):
            idx_fn, w_fn = _emit_grid_level(l, None, None, None, _GRID_RES[l])
            do_level(2 * l, (xa_v, xb_v, xc_v),
                     lambda *xs, f=idx_fn: f(None, *xs), w_fn, 8)

        plane_coords = [(xa_v, xb_v), (xb_v, xc_v), (xc_v, xa_v)]
        for p in range(3):
            for q in range(4):
                idx_fn, w_fn = _emit_plane_level(p, q)
                do_level(32 + (p * 4 + q) * 2, plane_coords[p], idx_fn, w_fn, 4)

        pltpu.sync_copy(acc_v, enc_hbm.at[:, pl.ds(base, _C)])
        return ()

    lax.fori_loop(0, _NCH, chunk, ())


def _mlp_body(enc_ref, zt_ref, w0a_ref, w0b_ref, w1_ref, w2_ref, out_ref):
    pe = jnp.sin(zt_ref[...])
    h = jnp.dot(w0a_ref[...], enc_ref[...], preferred_element_type=jnp.float32)
    h = h + jnp.dot(w0b_ref[...], pe, preferred_element_type=jnp.float32)
    h = jnp.maximum(h, 0.0)
    h = jnp.dot(w1_ref[...], h, preferred_element_type=jnp.float32)
    h = jnp.maximum(h, 0.0)
    out_ref[...] = jnp.dot(w2_ref[...], h, preferred_element_type=jnp.float32)


def _mlp(encT, zt, W0aT, W0bT, W1T, W2T):
    n = encT.shape[1]
    B = 2048
    return pl.pallas_call(
        _mlp_body,
        grid=(n // B,),
        in_specs=[
            pl.BlockSpec((56, B), lambda i: (0, i)),
            pl.BlockSpec((24, B), lambda i: (0, i)),
            pl.BlockSpec((64, 56), lambda i: (0, 0)),
            pl.BlockSpec((64, 24), lambda i: (0, 0)),
            pl.BlockSpec((64, 64), lambda i: (0, 0)),
            pl.BlockSpec((16, 64), lambda i: (0, 0)),
        ],
        out_specs=pl.BlockSpec((16, B), lambda i: (0, i)),
        out_shape=jax.ShapeDtypeStruct((16, n), jnp.float32),
    )(encT, zt, W0aT, W0bT, W1T, W2T)


def kernel(in_tensor, grid_table, plane0, plane1, plane2, W0, W1, W2):
    x0 = in_tensor[:, 0]
    x1 = in_tensor[:, 1]
    x2 = in_tensor[:, 2]
    flat = jnp.concatenate([
        grid_table.reshape(-1), plane0.reshape(-1),
        plane1.reshape(-1), plane2.reshape(-1)])
    encT = _make_sc_encode()(x0, x1, x2, flat)

    xT = in_tensor.T
    tiled = jnp.tile(xT, (4, 1))
    scales = jnp.repeat(jnp.asarray([1.0, 2.0, 4.0, 8.0], jnp.float32), 3)[:, None]
    z12 = tiled * scales
    zt = jnp.concatenate([z12, z12 + jnp.float32(0.5 * math.pi)], axis=0)

    outT = _mlp(encT, zt, W0[:56].T, W0[56:].T, W1.T, W2.T)
    return outT.T


# speedup vs baseline: 43.5296x; 2.0777x over previous
import functools
import math

import jax
import jax.numpy as jnp
import numpy as np
from jax import lax
from jax.experimental import pallas as pl
from jax.experimental.pallas import tpu as pltpu
import jax.experimental.pallas.tpu_sc as plsc

_N = 524288
_GRID_LEVELS = 16
_GRID_LOG2_T = 19
_GRID_BASE = 16
_MAX_RES = 1024
_GRID_PLS = float(np.exp((np.log(_MAX_RES) - np.log(_GRID_BASE)) / (_GRID_LEVELS - 1)))
_PLANE_LOG2_T = 17
_PLANE_BASE = _MAX_RES // 4
_POS_DEG = 4
_P1 = np.int32(np.uint32(2654435761).astype(np.int32))
_P2 = np.int32(805459861)

_NW = 32
_PPW = _N // _NW
_C = 512
_STEPS = _C // 16
_NCH = _PPW // _C
_SUB = 2048

_GRID_RES = [int(np.floor(_GRID_BASE * (_GRID_PLS ** l))) for l in range(_GRID_LEVELS)]
_GRID_T = 1 << _GRID_LOG2_T
_PLANE_T = 1 << _PLANE_LOG2_T
_PLANE_RES = [_PLANE_BASE * (2 ** q) for q in range(4)]
_FLAT_GRID_OFF = 0
_FLAT_PLANE_OFF = _GRID_LEVELS * _GRID_T


def _emit_grid_level(l, xa, xb, xc, res):
    S = res + 1
    dense = S ** 3 <= _GRID_T
    row_off = l * _GRID_T

    def indices(j0, xav, xbv, xcv):
        rf = jnp.float32(res)
        fa = xav * rf
        fb = xbv * rf
        fc = xcv * rf
        ia = fa.astype(jnp.int32)
        ib = fb.astype(jnp.int32)
        ic = fc.astype(jnp.int32)
        lvl_off = np.int32(2 * _GRID_T * l)
        out = []
        if dense:
            base = ia + ib * np.int32(S) + ic * np.int32(S * S)
            for k in range(8):
                b0, b1, b2 = k & 1, (k >> 1) & 1, (k >> 2) & 1
                m = base + np.int32(b0 + b1 * S + b2 * S * S)
                e0 = m + m - (m & np.int32(127)) + lvl_off
                out.append((e0, e0 + np.int32(128)))
        else:
            h1 = ib * _P1
            h1p = h1 + _P1
            h2 = ic * _P2
            h2p = h2 + _P2
            iap = ia + np.int32(1)
            msk = np.int32(_GRID_T - 1)
            for k in range(8):
                b0, b1, b2 = k & 1, (k >> 1) & 1, (k >> 2) & 1
                t = (iap if b0 else ia) ^ (h1p if b1 else h1) ^ (h2p if b2 else h2)
                m = t & msk
                e0 = m + m - (m & np.int32(127)) + lvl_off
                out.append((e0, e0 + np.int32(128)))
        return out

    def weights(xav, xbv, xcv):
        rf = jnp.float32(res)
        fa = xav * rf
        fb = xbv * rf
        fc = xcv * rf
        fra = fa - fa.astype(jnp.int32).astype(jnp.float32)
        frb = fb - fb.astype(jnp.int32).astype(jnp.float32)
        frc = fc - fc.astype(jnp.int32).astype(jnp.float32)
        wa = (jnp.float32(1.0) - fra, fra)
        wb = (jnp.float32(1.0) - frb, frb)
        wc = (jnp.float32(1.0) - frc, frc)
        ws = []
        for k in range(8):
            b0, b1, b2 = k & 1, (k >> 1) & 1, (k >> 2) & 1
            ws.append(wa[b0] * wb[b1] * wc[b2])
        return ws

    return indices, weights


def _emit_plane_level(p, q):
    res = _PLANE_RES[q]
    S = res + 1
    dense = S * S <= _PLANE_T
    row_off = _FLAT_PLANE_OFF + (p * 4 + q) * _PLANE_T

    def indices(xav, xbv):
        rf = jnp.float32(res)
        fa = xav * rf
        fb = xbv * rf
        ia = fa.astype(jnp.int32)
        ib = fb.astype(jnp.int32)
        lvl_off = np.int32(2 * _PLANE_T * q)
        out = []
        if dense:
            base = ia + ib * np.int32(S)
            for k in range(4):
                b0, b1 = k & 1, (k >> 1) & 1
                m = base + np.int32(b0 + b1 * S)
                e0 = m + m - (m & np.int32(127)) + lvl_off
                out.append((e0, e0 + np.int32(128)))
        else:
            h1 = ib * _P1
            h1p = h1 + _P1
            iap = ia + np.int32(1)
            msk = np.int32(_PLANE_T - 1)
            for k in range(4):
                b0, b1 = k & 1, (k >> 1) & 1
                t = (iap if b0 else ia) ^ (h1p if b1 else h1)
                m = t & msk
                e0 = m + m - (m & np.int32(127)) + lvl_off
                out.append((e0, e0 + np.int32(128)))
        return out

    def weights(xav, xbv):
        rf = jnp.float32(res)
        fa = xav * rf
        fb = xbv * rf
        fra = fa - fa.astype(jnp.int32).astype(jnp.float32)
        frb = fb - fb.astype(jnp.int32).astype(jnp.float32)
        wa = (jnp.float32(1.0) - fra, fra)
        wb = (jnp.float32(1.0) - frb, frb)
        return [wa[k & 1] * wb[(k >> 1) & 1] for k in range(4)]

    return indices, weights


@functools.cache
def _make_sc_encode():
    mesh = plsc.VectorSubcoreMesh(
        core_axis_name="c", subcore_axis_name="s", num_cores=2, num_subcores=16)
    return pl.kernel(
        _sc_encode_body,
        out_type=jax.ShapeDtypeStruct((56, _N), jnp.float32),
        mesh=mesh,
        scratch_types=[
            pltpu.VMEM((_C,), jnp.float32),
            pltpu.VMEM((_C,), jnp.float32),
            pltpu.VMEM((_C,), jnp.float32),
            pltpu.VMEM((16 * _C,), jnp.int32),
            pltpu.VMEM((16 * _C,), jnp.float32),
            pltpu.VMEM((56, _C), jnp.float32),
            pltpu.SemaphoreType.DMA,
        ],
    )


def _sc_encode_body(x0_hbm, x1_hbm, x2_hbm, fg_hbm, fp0_hbm, fp1_hbm, fp2_hbm,
                    enc_hbm, xa_v, xb_v, xc_v, idx_v, dst_v, acc_v, sem):
    wid = lax.axis_index("s") * 2 + lax.axis_index("c")

    def chunk(g, _):
        base = wid * _PPW + g * _C
        pltpu.sync_copy(x0_hbm.at[pl.ds(base, _C)], xa_v)
        pltpu.sync_copy(x1_hbm.at[pl.ds(base, _C)], xb_v)
        pltpu.sync_copy(x2_hbm.at[pl.ds(base, _C)], xc_v)

        def do_level(row0, xrefs, idx_fn, w_fn, ncorner, tbl_hbm):
            nidx = 2 * ncorner * _C

            def p1(step, _):
                j0 = step * 16
                xs = [r[pl.ds(j0, 16)] for r in xrefs]
                pairs = idx_fn(*xs)
                for k, (e0, e1) in enumerate(pairs):
                    idx_v[pl.ds((2 * k) * _C + j0, 16)] = e0
                    idx_v[pl.ds((2 * k + 1) * _C + j0, 16)] = e1
                return ()

            lax.fori_loop(0, _STEPS, p1, ())
            for u in range(nidx // _SUB):
                pltpu.async_copy(
                    tbl_hbm.at[idx_v.at[pl.ds(u * _SUB, _SUB)]],
                    dst_v.at[pl.ds(u * _SUB, _SUB)], sem).wait()

            def p2(step, _):
                j0 = step * 16
                xs = [r[pl.ds(j0, 16)] for r in xrefs]
                ws = w_fn(*xs)
                acc0 = None
                acc1 = None
                for k in range(ncorner):
                    f0 = dst_v[pl.ds((2 * k) * _C + j0, 16)]
                    f1 = dst_v[pl.ds((2 * k + 1) * _C + j0, 16)]
                    w = ws[k]
                    if acc0 is None:
                        acc0 = w * f0
                        acc1 = w * f1
                    else:
                        acc0 = acc0 + w * f0
                        acc1 = acc1 + w * f1
                acc_v[row0, pl.ds(j0, 16)] = acc0
                acc_v[row0 + 1, pl.ds(j0, 16)] = acc1
                return ()

            lax.fori_loop(0, _STEPS, p2, ())

        for l in range(_GRID_LEVELS):
            idx_fn, w_fn = _emit_grid_level(l, None, None, None, _GRID_RES[l])
            do_level(2 * l, (xa_v, xb_v, xc_v),
                     lambda *xs, f=idx_fn: f(None, *xs), w_fn, 8, fg_hbm)

        plane_coords = [(xa_v, xb_v), (xb_v, xc_v), (xc_v, xa_v)]
        plane_tbls = [fp0_hbm, fp1_hbm, fp2_hbm]
        for p in range(3):
            for q in range(4):
                idx_fn, w_fn = _emit_plane_level(p, q)
                do_level(32 + (p * 4 + q) * 2, plane_coords[p], idx_fn, w_fn,
                         4, plane_tbls[p])

        pltpu.sync_copy(acc_v, enc_hbm.at[:, pl.ds(base, _C)])
        return ()

    lax.fori_loop(0, _NCH, chunk, ())


def _mlp_body(enc_ref, zt_ref, w0a_ref, w0b_ref, w1_ref, w2_ref, out_ref):
    pe = jnp.sin(zt_ref[...])
    h = jnp.dot(w0a_ref[...], enc_ref[...], preferred_element_type=jnp.float32)
    h = h + jnp.dot(w0b_ref[...], pe, preferred_element_type=jnp.float32)
    h = jnp.maximum(h, 0.0)
    h = jnp.dot(w1_ref[...], h, preferred_element_type=jnp.float32)
    h = jnp.maximum(h, 0.0)
    out_ref[...] = jnp.dot(w2_ref[...], h, preferred_element_type=jnp.float32)


def _mlp(encT, zt, W0aT, W0bT, W1T, W2T):
    n = encT.shape[1]
    B = 2048
    return pl.pallas_call(
        _mlp_body,
        grid=(n // B,),
        in_specs=[
            pl.BlockSpec((56, B), lambda i: (0, i)),
            pl.BlockSpec((24, B), lambda i: (0, i)),
            pl.BlockSpec((64, 56), lambda i: (0, 0)),
            pl.BlockSpec((64, 24), lambda i: (0, 0)),
            pl.BlockSpec((64, 64), lambda i: (0, 0)),
            pl.BlockSpec((16, 64), lambda i: (0, 0)),
        ],
        out_specs=pl.BlockSpec((16, B), lambda i: (0, i)),
        out_shape=jax.ShapeDtypeStruct((16, n), jnp.float32),
    )(encT, zt, W0aT, W0bT, W1T, W2T)


def kernel(in_tensor, grid_table, plane0, plane1, plane2, W0, W1, W2):
    x0 = in_tensor[:, 0]
    x1 = in_tensor[:, 1]
    x2 = in_tensor[:, 2]
    def tiled_flat(tbl):
        lv, t, _ = tbl.shape
        return tbl.reshape(lv, t // 128, 128, 2).transpose(0, 1, 3, 2).reshape(-1)

    fg = tiled_flat(grid_table)
    fp0 = tiled_flat(plane0)
    fp1 = tiled_flat(plane1)
    fp2 = tiled_flat(plane2)
    encT = _make_sc_encode()(x0, x1, x2, fg, fp0, fp1, fp2)

    xT = in_tensor.T
    tiled = jnp.tile(xT, (4, 1))
    scales = jnp.repeat(jnp.asarray([1.0, 2.0, 4.0, 8.0], jnp.float32), 3)[:, None]
    z12 = tiled * scales
    zt = jnp.concatenate([z12, z12 + jnp.float32(0.5 * math.pi)], axis=0)

    outT = _mlp(encT, zt, W0[:56].T, W0[56:].T, W1.T, W2.T)
    return outT.T


# speedup vs baseline: 61.4294x; 1.4112x over previous
import functools
import math

import jax
import jax.numpy as jnp
import numpy as np
from jax import lax
from jax.experimental import pallas as pl
from jax.experimental.pallas import tpu as pltpu
import jax.experimental.pallas.tpu_sc as plsc

_N = 524288
_GRID_LEVELS = 16
_GRID_LOG2_T = 19
_GRID_BASE = 16
_MAX_RES = 1024
_GRID_PLS = float(np.exp((np.log(_MAX_RES) - np.log(_GRID_BASE)) / (_GRID_LEVELS - 1)))
_PLANE_LOG2_T = 17
_PLANE_BASE = _MAX_RES // 4
_POS_DEG = 4
_P1 = np.int32(np.uint32(2654435761).astype(np.int32))
_P2 = np.int32(805459861)

_NW = 32
_PPW = _N // _NW
_C = 512
_STEPS = _C // 16
_NCH = _PPW // _C
_SUB = 2048

_GRID_RES = [int(np.floor(_GRID_BASE * (_GRID_PLS ** l))) for l in range(_GRID_LEVELS)]
_GRID_T = 1 << _GRID_LOG2_T
_PLANE_T = 1 << _PLANE_LOG2_T
_PLANE_RES = [_PLANE_BASE * (2 ** q) for q in range(4)]
_FLAT_GRID_OFF = 0
_FLAT_PLANE_OFF = _GRID_LEVELS * _GRID_T


def _emit_grid_level(l, xa, xb, xc, res):
    S = res + 1
    dense = S ** 3 <= _GRID_T
    row_off = l * _GRID_T

    def indices(j0, xav, xbv, xcv):
        rf = jnp.float32(res)
        fa = xav * rf
        fb = xbv * rf
        fc = xcv * rf
        ia = fa.astype(jnp.int32)
        ib = fb.astype(jnp.int32)
        ic = fc.astype(jnp.int32)
        lvl_off = np.int32(2 * _GRID_T * l)
        out = []
        if dense:
            base = ia + ib * np.int32(S) + ic * np.int32(S * S)
            for k in range(8):
                b0, b1, b2 = k & 1, (k >> 1) & 1, (k >> 2) & 1
                m = base + np.int32(b0 + b1 * S + b2 * S * S)
                e0 = m + m - (m & np.int32(127)) + lvl_off
                out.append((e0, e0 + np.int32(128)))
        else:
            h1 = ib * _P1
            h1p = h1 + _P1
            h2 = ic * _P2
            h2p = h2 + _P2
            iap = ia + np.int32(1)
            msk = np.int32(_GRID_T - 1)
            for k in range(8):
                b0, b1, b2 = k & 1, (k >> 1) & 1, (k >> 2) & 1
                t = (iap if b0 else ia) ^ (h1p if b1 else h1) ^ (h2p if b2 else h2)
                m = t & msk
                e0 = m + m - (m & np.int32(127)) + lvl_off
                out.append((e0, e0 + np.int32(128)))
        return out

    def weights(xav, xbv, xcv):
        rf = jnp.float32(res)
        fa = xav * rf
        fb = xbv * rf
        fc = xcv * rf
        fra = fa - fa.astype(jnp.int32).astype(jnp.float32)
        frb = fb - fb.astype(jnp.int32).astype(jnp.float32)
        frc = fc - fc.astype(jnp.int32).astype(jnp.float32)
        wa = (jnp.float32(1.0) - fra, fra)
        wb = (jnp.float32(1.0) - frb, frb)
        wc = (jnp.float32(1.0) - frc, frc)
        ws = []
        for k in range(8):
            b0, b1, b2 = k & 1, (k >> 1) & 1, (k >> 2) & 1
            ws.append(wa[b0] * wb[b1] * wc[b2])
        return ws

    return indices, weights


def _emit_plane_level(p, q):
    res = _PLANE_RES[q]
    S = res + 1
    dense = S * S <= _PLANE_T
    row_off = _FLAT_PLANE_OFF + (p * 4 + q) * _PLANE_T

    def indices(xav, xbv):
        rf = jnp.float32(res)
        fa = xav * rf
        fb = xbv * rf
        ia = fa.astype(jnp.int32)
        ib = fb.astype(jnp.int32)
        lvl_off = np.int32(2 * _PLANE_T * q)
        out = []
        if dense:
            base = ia + ib * np.int32(S)
            for k in range(4):
                b0, b1 = k & 1, (k >> 1) & 1
                m = base + np.int32(b0 + b1 * S)
                e0 = m + m - (m & np.int32(127)) + lvl_off
                out.append((e0, e0 + np.int32(128)))
        else:
            h1 = ib * _P1
            h1p = h1 + _P1
            iap = ia + np.int32(1)
            msk = np.int32(_PLANE_T - 1)
            for k in range(4):
                b0, b1 = k & 1, (k >> 1) & 1
                t = (iap if b0 else ia) ^ (h1p if b1 else h1)
                m = t & msk
                e0 = m + m - (m & np.int32(127)) + lvl_off
                out.append((e0, e0 + np.int32(128)))
        return out

    def weights(xav, xbv):
        rf = jnp.float32(res)
        fa = xav * rf
        fb = xbv * rf
        fra = fa - fa.astype(jnp.int32).astype(jnp.float32)
        frb = fb - fb.astype(jnp.int32).astype(jnp.float32)
        wa = (jnp.float32(1.0) - fra, fra)
        wb = (jnp.float32(1.0) - frb, frb)
        return [wa[k & 1] * wb[(k >> 1) & 1] for k in range(4)]

    return indices, weights


@functools.cache
def _make_sc_encode():
    mesh = plsc.VectorSubcoreMesh(
        core_axis_name="c", subcore_axis_name="s", num_cores=2, num_subcores=16)
    return pl.kernel(
        _sc_encode_body,
        out_type=jax.ShapeDtypeStruct((56, _N), jnp.float32),
        mesh=mesh,
        scratch_types=[
            pltpu.VMEM((_C,), jnp.float32),
            pltpu.VMEM((_C,), jnp.float32),
            pltpu.VMEM((_C,), jnp.float32),
            pltpu.VMEM((16 * _C,), jnp.int32),
            pltpu.VMEM((16 * _C,), jnp.int32),
            pltpu.VMEM((16 * _C,), jnp.float32),
            pltpu.VMEM((16 * _C,), jnp.float32),
            pltpu.VMEM((56, _C), jnp.float32),
            pltpu.SemaphoreType.DMA,
            pltpu.SemaphoreType.DMA,
        ],
    )


def _sc_encode_body(x0_hbm, x1_hbm, x2_hbm, fg_hbm, fp0_hbm, fp1_hbm, fp2_hbm,
                    enc_hbm, xa_v, xb_v, xc_v, idx0_v, idx1_v, dst0_v, dst1_v,
                    acc_v, sem0, sem1):
    wid = lax.axis_index("s") * 2 + lax.axis_index("c")
    idx_bufs = (idx0_v, idx1_v)
    dst_bufs = (dst0_v, dst1_v)
    sems = (sem0, sem1)

    def chunk(g, _):
        base = wid * _PPW + g * _C
        pltpu.sync_copy(x0_hbm.at[pl.ds(base, _C)], xa_v)
        pltpu.sync_copy(x1_hbm.at[pl.ds(base, _C)], xb_v)
        pltpu.sync_copy(x2_hbm.at[pl.ds(base, _C)], xc_v)

        stages = []
        for l in range(_GRID_LEVELS):
            idx_fn, w_fn = _emit_grid_level(l, None, None, None, _GRID_RES[l])
            stages.append((2 * l, (xa_v, xb_v, xc_v),
                           (lambda *xs, f=idx_fn: f(None, *xs)), w_fn, 8,
                           fg_hbm))
        plane_coords = [(xa_v, xb_v), (xb_v, xc_v), (xc_v, xa_v)]
        plane_tbls = [fp0_hbm, fp1_hbm, fp2_hbm]
        for p in range(3):
            for q in range(4):
                idx_fn, w_fn = _emit_plane_level(p, q)
                stages.append((32 + (p * 4 + q) * 2, plane_coords[p],
                               idx_fn, w_fn, 4, plane_tbls[p]))

        def pass1(st, b):
            row0, xrefs, idx_fn, w_fn, ncorner, tbl = st
            idx_v = idx_bufs[b]

            def p1(step, _):
                j0 = step * 16
                xs = [r[pl.ds(j0, 16)] for r in xrefs]
                pairs = idx_fn(*xs)
                for k, (e0, e1) in enumerate(pairs):
                    idx_v[pl.ds((2 * k) * _C + j0, 16)] = e0
                    idx_v[pl.ds((2 * k + 1) * _C + j0, 16)] = e1
                return ()

            lax.fori_loop(0, _STEPS, p1, ())

        def fire(st, b):
            row0, xrefs, idx_fn, w_fn, ncorner, tbl = st
            nidx = 2 * ncorner * _C
            descs = []
            for u in range(nidx // _SUB):
                descs.append(pltpu.async_copy(
                    tbl.at[idx_bufs[b].at[pl.ds(u * _SUB, _SUB)]],
                    dst_bufs[b].at[pl.ds(u * _SUB, _SUB)], sems[b]))
            return descs

        def pass2(st, b):
            row0, xrefs, idx_fn, w_fn, ncorner, tbl = st
            dst_v = dst_bufs[b]

            def p2(step, _):
                j0 = step * 16
                xs = [r[pl.ds(j0, 16)] for r in xrefs]
                ws = w_fn(*xs)
                acc0 = None
                acc1 = None
                for k in range(ncorner):
                    f0 = dst_v[pl.ds((2 * k) * _C + j0, 16)]
                    f1 = dst_v[pl.ds((2 * k + 1) * _C + j0, 16)]
                    w = ws[k]
                    if acc0 is None:
                        acc0 = w * f0
                        acc1 = w * f1
                    else:
                        acc0 = acc0 + w * f0
                        acc1 = acc1 + w * f1
                acc_v[row0, pl.ds(j0, 16)] = acc0
                acc_v[row0 + 1, pl.ds(j0, 16)] = acc1
                return ()

            lax.fori_loop(0, _STEPS, p2, ())

        pending = None
        for i, st in enumerate(stages):
            b = i % 2
            pass1(st, b)
            d = fire(st, b)
            if pending is not None:
                for dd in pending:
                    dd.wait()
                pass2(stages[i - 1], 1 - b)
            pending = d
        for dd in pending:
            dd.wait()
        pass2(stages[-1], (len(stages) - 1) % 2)

        pltpu.sync_copy(acc_v, enc_hbm.at[:, pl.ds(base, _C)])
        return ()

    lax.fori_loop(0, _NCH, chunk, ())


def _mlp_body(enc_ref, zt_ref, w0a_ref, w0b_ref, w1_ref, w2_ref, out_ref):
    pe = jnp.sin(zt_ref[...])
    h = jnp.dot(w0a_ref[...], enc_ref[...], preferred_element_type=jnp.float32)
    h = h + jnp.dot(w0b_ref[...], pe, preferred_element_type=jnp.float32)
    h = jnp.maximum(h, 0.0)
    h = jnp.dot(w1_ref[...], h, preferred_element_type=jnp.float32)
    h = jnp.maximum(h, 0.0)
    out_ref[...] = jnp.dot(w2_ref[...], h, preferred_element_type=jnp.float32)


def _mlp(encT, zt, W0aT, W0bT, W1T, W2T):
    n = encT.shape[1]
    B = 2048
    return pl.pallas_call(
        _mlp_body,
        grid=(n // B,),
        in_specs=[
            pl.BlockSpec((56, B), lambda i: (0, i)),
            pl.BlockSpec((24, B), lambda i: (0, i)),
            pl.BlockSpec((64, 56), lambda i: (0, 0)),
            pl.BlockSpec((64, 24), lambda i: (0, 0)),
            pl.BlockSpec((64, 64), lambda i: (0, 0)),
            pl.BlockSpec((16, 64), lambda i: (0, 0)),
        ],
        out_specs=pl.BlockSpec((16, B), lambda i: (0, i)),
        out_shape=jax.ShapeDtypeStruct((16, n), jnp.float32),
    )(encT, zt, W0aT, W0bT, W1T, W2T)


def kernel(in_tensor, grid_table, plane0, plane1, plane2, W0, W1, W2):
    x0 = in_tensor[:, 0]
    x1 = in_tensor[:, 1]
    x2 = in_tensor[:, 2]
    def tiled_flat(tbl):
        lv, t, _ = tbl.shape
        return tbl.reshape(lv, t // 128, 128, 2).transpose(0, 1, 3, 2).reshape(-1)

    fg = tiled_flat(grid_table)
    fp0 = tiled_flat(plane0)
    fp1 = tiled_flat(plane1)
    fp2 = tiled_flat(plane2)
    encT = _make_sc_encode()(x0, x1, x2, fg, fp0, fp1, fp2)

    xT = in_tensor.T
    tiled = jnp.tile(xT, (4, 1))
    scales = jnp.repeat(jnp.asarray([1.0, 2.0, 4.0, 8.0], jnp.float32), 3)[:, None]
    z12 = tiled * scales
    zt = jnp.concatenate([z12, z12 + jnp.float32(0.5 * math.pi)], axis=0)

    outT = _mlp(encT, zt, W0[:56].T, W0[56:].T, W1.T, W2.T)
    return outT.T
